# Initial kernel scaffold; baseline (speedup 1.0000x reference)
import jax
import jax.numpy as jnp
from jax.experimental import pallas as pl


def kernel(x, gamma, beta, wg, W1, b1, W2, b2):
    raise NotImplementedError("write your pallas kernel here")


# speedup vs baseline: 3.4298x; 3.4298x over previous
import functools

import jax
import jax.numpy as jnp
from jax import lax
from jax.experimental import pallas as pl
from jax.experimental.pallas import tpu as pltpu
from jax.experimental.pallas import tpu_sc as plsc

TOKENS = 2048
HID = 1024
FF = 4096
NE = 8
CAP = 640
NSLOT = NE * CAP
TRASH = NSLOT
NROWS = NSLOT + 8
NPAIR = 2 * TOKENS

SC_CORES = 2
SC_SUBCORES = 16
NWORK = SC_CORES * SC_SUBCORES
PAIRS_PER_W = NPAIR // NWORK
TOKS_PER_W = TOKENS // NWORK
CHUNK = 64


def _routing_body(x_ref, gamma_ref, beta_ref, wgt_ref,
                  flat_ref, slots_ref, w_ref, aux_ref, counts_ref):
    x = x_ref[...]
    mu = jnp.mean(x, axis=1, keepdims=True)
    xc = x - mu
    var = jnp.mean(xc * xc, axis=1, keepdims=True)
    flat = xc / jnp.sqrt(var + 1e-5) * gamma_ref[...] + beta_ref[...]
    flat_ref[...] = flat

    logits = jnp.dot(flat, wgt_ref[...], preferred_element_type=jnp.float32)
    mx = jnp.max(logits, axis=1, keepdims=True)
    eg = jnp.exp(logits - mx)
    gates = eg / jnp.sum(eg, axis=1, keepdims=True)

    colid = lax.broadcasted_iota(jnp.int32, (TOKENS, NE), 1)
    g0 = jnp.max(gates, axis=1, keepdims=True)
    idx0 = jnp.min(jnp.where(gates >= g0, colid, NE), axis=1, keepdims=True)
    m0 = colid == idx0
    gates_m = jnp.where(m0, jnp.float32(-1e30), gates)
    g1 = jnp.max(gates_m, axis=1, keepdims=True)
    idx1 = jnp.min(jnp.where(gates_m >= g1, colid, NE), axis=1, keepdims=True)
    m1 = colid == idx1

    def cumsum_tokens(m):
        s = m.astype(jnp.float32)
        d = 1
        while d < TOKENS:
            z = jnp.zeros((d, NE), dtype=jnp.float32)
            s = s + jnp.concatenate([z, s[:TOKENS - d, :]], axis=0)
            d *= 2
        return s

    c0 = cumsum_tokens(m0)
    loc0 = c0 - 1.0
    kept0 = m0 & (loc0 < CAP)
    used0 = jnp.sum(kept0.astype(jnp.float32), axis=0, keepdims=True)
    c1 = cumsum_tokens(m1)
    loc1 = c1 - 1.0 + used0
    kept1 = m1 & (loc1 < CAP)
    used1 = jnp.sum(kept1.astype(jnp.float32), axis=0, keepdims=True)

    k0 = jnp.max(kept0.astype(jnp.float32), axis=1, keepdims=True)
    k1 = jnp.max(kept1.astype(jnp.float32), axis=1, keepdims=True)
    gate0 = g0 * k0
    gate1 = g1 * k1
    denom = jnp.maximum(gate0 + gate1, 1e-9)
    w0 = gate0 / denom
    w1 = gate1 / denom
    w_ref[...] = jnp.concatenate([w0, w1], axis=1)

    loc0_t = jnp.sum(jnp.where(kept0, loc0, 0.0), axis=1, keepdims=True)
    loc1_t = jnp.sum(jnp.where(kept1, loc1, 0.0), axis=1, keepdims=True)
    slot0 = jnp.where(k0 > 0.0, idx0 * CAP + loc0_t.astype(jnp.int32), TRASH)
    slot1 = jnp.where(k1 > 0.0, idx1 * CAP + loc1_t.astype(jnp.int32), TRASH)
    slots_ref[...] = jnp.concatenate([slot0, slot1], axis=1)

    me = jnp.mean(gates, axis=0, keepdims=True)
    ce = used0 / jnp.float32(TOKENS)
    aux_ref[...] = jnp.sum(me * ce, axis=1, keepdims=True) * jnp.float32(NE)
    counts_ref[...] = used0 + used1


def _routing_call(x2, gamma2, beta2, wgt):
    return pl.pallas_call(
        _routing_body,
        out_shape=(
            jax.ShapeDtypeStruct((TOKENS, HID), jnp.float32),
            jax.ShapeDtypeStruct((TOKENS, 2), jnp.int32),
            jax.ShapeDtypeStruct((TOKENS, 2), jnp.float32),
            jax.ShapeDtypeStruct((1, 1), jnp.float32),
            jax.ShapeDtypeStruct((1, NE), jnp.float32),
        ),
    )(x2, gamma2, beta2, wgt)


def _dispatch_body(flat_hbm, slots_hbm, xdisp_hbm, idx_v, rows_v, sem):
    wid = lax.axis_index("s") * SC_CORES + lax.axis_index("c")
    base = wid * PAIRS_PER_W
    for c in range(PAIRS_PER_W // CHUNK):
        off = pl.multiple_of(base + c * CHUNK, CHUNK)
        roff = pl.multiple_of(lax.rem(off, TOKENS), CHUNK)
        pltpu.sync_copy(slots_hbm.at[pl.ds(off, CHUNK)], idx_v)
        pltpu.sync_copy(flat_hbm.at[pl.ds(roff, CHUNK)], rows_v)
        pltpu.async_copy(rows_v, xdisp_hbm.at[idx_v], sem).wait()


def _dispatch_call(flat, slots):
    mesh = plsc.VectorSubcoreMesh(core_axis_name="c", subcore_axis_name="s",
                                  num_cores=SC_CORES, num_subcores=SC_SUBCORES)
    fn = pl.kernel(
        _dispatch_body,
        mesh=mesh,
        out_type=jax.ShapeDtypeStruct((NROWS, HID), jnp.float32),
        scratch_types=[
            pltpu.VMEM((CHUNK,), jnp.int32),
            pltpu.VMEM((CHUNK, HID), jnp.float32),
            pltpu.SemaphoreType.DMA,
        ],
    )
    return fn(flat, slots)


FTS = 512
FT = FF // FTS


def _ffn_body(x_ref, w1_ref, b1_ref, w2_ref, b2_ref, y_ref):
    f = pl.program_id(1)
    x = x_ref[...]
    h = lax.dot_general(x, w1_ref[0], (((1,), (1,)), ((), ())),
                        preferred_element_type=jnp.float32)
    h = h + b1_ref[0]
    h = 0.5 * h * (1.0 + lax.erf(h * jnp.float32(0.7071067811865476)))
    part = lax.dot_general(h, w2_ref[0], (((1,), (1,)), ((), ())),
                           preferred_element_type=jnp.float32)

    @pl.when(f == 0)
    def _():
        y_ref[...] = part + b2_ref[0]

    @pl.when(f != 0)
    def _():
        y_ref[...] = y_ref[...] + part


def _ffn_call(xdisp, W1, b1, W2, b2):
    return pl.pallas_call(
        _ffn_body,
        grid=(NE, FT),
        in_specs=[
            pl.BlockSpec((CAP, HID), lambda e, f: (e, 0)),
            pl.BlockSpec((1, FTS, HID), lambda e, f: (e, f, 0)),
            pl.BlockSpec((1, 1, FTS), lambda e, f: (e * FT + f, 0, 0)),
            pl.BlockSpec((1, HID, FTS), lambda e, f: (e, 0, f)),
            pl.BlockSpec((1, 1, HID), lambda e, f: (e, 0, 0)),
        ],
        out_specs=pl.BlockSpec((CAP, HID), lambda e, f: (e, 0)),
        out_shape=jax.ShapeDtypeStruct((NROWS, HID), jnp.float32),
    )(xdisp, W1, b1.reshape(NE * FT, 1, FTS), W2, b2.reshape(NE, 1, HID))


def _gather_body(y_hbm, slots_hbm, y0_hbm, y1_hbm, idx_v, rows_v, sem):
    wid = lax.axis_index("s") * SC_CORES + lax.axis_index("c")
    base = pl.multiple_of(wid * TOKS_PER_W, TOKS_PER_W)
    pltpu.sync_copy(slots_hbm.at[pl.ds(base, TOKS_PER_W)], idx_v)
    pltpu.async_copy(y_hbm.at[idx_v], rows_v, sem).wait()
    pltpu.sync_copy(rows_v, y0_hbm.at[pl.ds(base, TOKS_PER_W)])
    off1 = pl.multiple_of(TOKENS + base, TOKS_PER_W)
    pltpu.sync_copy(slots_hbm.at[pl.ds(off1, TOKS_PER_W)], idx_v)
    pltpu.async_copy(y_hbm.at[idx_v], rows_v, sem).wait()
    pltpu.sync_copy(rows_v, y1_hbm.at[pl.ds(base, TOKS_PER_W)])


def _gather_call(y, slots):
    mesh = plsc.VectorSubcoreMesh(core_axis_name="c", subcore_axis_name="s",
                                  num_cores=SC_CORES, num_subcores=SC_SUBCORES)
    fn = pl.kernel(
        _gather_body,
        mesh=mesh,
        out_type=(
            jax.ShapeDtypeStruct((TOKENS, HID), jnp.float32),
            jax.ShapeDtypeStruct((TOKENS, HID), jnp.float32),
        ),
        scratch_types=[
            pltpu.VMEM((TOKS_PER_W,), jnp.int32),
            pltpu.VMEM((TOKS_PER_W, HID), jnp.float32),
            pltpu.SemaphoreType.DMA,
        ],
    )
    return fn(y, slots)


TBLK = 256


def _combine_body(x_ref, w_ref, y0_ref, y1_ref, o_ref):
    w0 = w_ref[:, 0:1]
    w1 = w_ref[:, 1:2]
    o_ref[...] = x_ref[...] + w0 * y0_ref[...] + w1 * y1_ref[...]


def _combine_call(x2, w2d, y0, y1):
    return pl.pallas_call(
        _combine_body,
        grid=(TOKENS // TBLK,),
        in_specs=[
            pl.BlockSpec((TBLK, HID), lambda t: (t, 0)),
            pl.BlockSpec((TBLK, 2), lambda t: (t, 0)),
            pl.BlockSpec((TBLK, HID), lambda t: (t, 0)),
            pl.BlockSpec((TBLK, HID), lambda t: (t, 0)),
        ],
        out_specs=pl.BlockSpec((TBLK, HID), lambda t: (t, 0)),
        out_shape=jax.ShapeDtypeStruct((TOKENS, HID), jnp.float32),
    )(x2, w2d, y0, y1)


@jax.jit
def kernel(x, gamma, beta, wg, W1, b1, W2, b2):
    x2 = x.reshape(TOKENS, HID)
    flat, slots2, w2d, aux, counts = _routing_call(
        x2, gamma.reshape(1, HID), beta.reshape(1, HID), wg.T)
    slots = jnp.concatenate([slots2[:, 0], slots2[:, 1]])
    xdisp = _dispatch_call(flat, slots)
    y = _ffn_call(xdisp, W1, b1, W2, b2)
    y0, y1 = _gather_call(y, slots)
    out = _combine_call(x2, w2d, y0, y1)
    return out.reshape(x.shape), aux[0, 0], counts[0]


# speedup vs baseline: 3.8519x; 1.1231x over previous
import functools

import jax
import jax.numpy as jnp
from jax import lax
from jax.experimental import pallas as pl
from jax.experimental.pallas import tpu as pltpu
from jax.experimental.pallas import tpu_sc as plsc

TOKENS = 2048
HID = 1024
FF = 4096
NE = 8
CAP = 640
NSLOT = NE * CAP
TRASH = NSLOT
NROWS = NSLOT + 8
NPAIR = 2 * TOKENS

SC_CORES = 2
SC_SUBCORES = 16
NWORK = SC_CORES * SC_SUBCORES
PAIRS_PER_W = NPAIR // NWORK
TOKS_PER_W = TOKENS // NWORK
CHUNK = 64


def _routing_body(x_ref, gamma_ref, beta_ref, wgt_ref,
                  flat_ref, slots_ref, w_ref, aux_ref, counts_ref):
    x = x_ref[...]
    mu = jnp.mean(x, axis=1, keepdims=True)
    xc = x - mu
    var = jnp.mean(xc * xc, axis=1, keepdims=True)
    flat = xc / jnp.sqrt(var + 1e-5) * gamma_ref[...] + beta_ref[...]
    flat_ref[...] = flat

    logits = jnp.dot(flat, wgt_ref[...], preferred_element_type=jnp.float32)
    mx = jnp.max(logits, axis=1, keepdims=True)
    eg = jnp.exp(logits - mx)
    gates = eg / jnp.sum(eg, axis=1, keepdims=True)

    colid = lax.broadcasted_iota(jnp.int32, (TOKENS, NE), 1)
    g0 = jnp.max(gates, axis=1, keepdims=True)
    idx0 = jnp.min(jnp.where(gates >= g0, colid, NE), axis=1, keepdims=True)
    m0 = colid == idx0
    gates_m = jnp.where(m0, jnp.float32(-1e30), gates)
    g1 = jnp.max(gates_m, axis=1, keepdims=True)
    idx1 = jnp.min(jnp.where(gates_m >= g1, colid, NE), axis=1, keepdims=True)
    m1 = colid == idx1

    def cumsum_tokens(m):
        s = m.astype(jnp.float32)
        d = 1
        while d < TOKENS:
            z = jnp.zeros((d, NE), dtype=jnp.float32)
            s = s + jnp.concatenate([z, s[:TOKENS - d, :]], axis=0)
            d *= 2
        return s

    c0 = cumsum_tokens(m0)
    loc0 = c0 - 1.0
    kept0 = m0 & (loc0 < CAP)
    used0 = jnp.sum(kept0.astype(jnp.float32), axis=0, keepdims=True)
    c1 = cumsum_tokens(m1)
    loc1 = c1 - 1.0 + used0
    kept1 = m1 & (loc1 < CAP)
    used1 = jnp.sum(kept1.astype(jnp.float32), axis=0, keepdims=True)

    k0 = jnp.max(kept0.astype(jnp.float32), axis=1, keepdims=True)
    k1 = jnp.max(kept1.astype(jnp.float32), axis=1, keepdims=True)
    gate0 = g0 * k0
    gate1 = g1 * k1
    denom = jnp.maximum(gate0 + gate1, 1e-9)
    w0 = gate0 / denom
    w1 = gate1 / denom
    w_ref[...] = jnp.concatenate([w0, w1], axis=1)

    loc0_t = jnp.sum(jnp.where(kept0, loc0, 0.0), axis=1, keepdims=True)
    loc1_t = jnp.sum(jnp.where(kept1, loc1, 0.0), axis=1, keepdims=True)
    slot0 = jnp.where(k0 > 0.0, idx0 * CAP + loc0_t.astype(jnp.int32), TRASH)
    slot1 = jnp.where(k1 > 0.0, idx1 * CAP + loc1_t.astype(jnp.int32), TRASH)
    slots_ref[...] = jnp.concatenate([slot0, slot1], axis=1)

    me = jnp.mean(gates, axis=0, keepdims=True)
    ce = used0 / jnp.float32(TOKENS)
    aux_ref[...] = jnp.sum(me * ce, axis=1, keepdims=True) * jnp.float32(NE)
    counts_ref[...] = used0 + used1


def _routing_call(x2, gamma2, beta2, wgt):
    return pl.pallas_call(
        _routing_body,
        out_shape=(
            jax.ShapeDtypeStruct((TOKENS, HID), jnp.float32),
            jax.ShapeDtypeStruct((TOKENS, 2), jnp.int32),
            jax.ShapeDtypeStruct((TOKENS, 2), jnp.float32),
            jax.ShapeDtypeStruct((1, 1), jnp.float32),
            jax.ShapeDtypeStruct((1, NE), jnp.float32),
        ),
    )(x2, gamma2, beta2, wgt)


def _dispatch_body(flat_hbm, slots_hbm, xdisp_hbm, idx_v, rows_v, sem):
    wid = lax.axis_index("s") * SC_CORES + lax.axis_index("c")
    base = wid * PAIRS_PER_W
    for c in range(PAIRS_PER_W // CHUNK):
        off = pl.multiple_of(base + c * CHUNK, CHUNK)
        roff = pl.multiple_of(lax.rem(off, TOKENS), CHUNK)
        pltpu.sync_copy(slots_hbm.at[pl.ds(off, CHUNK)], idx_v)
        pltpu.sync_copy(flat_hbm.at[pl.ds(roff, CHUNK)], rows_v)
        pltpu.async_copy(rows_v, xdisp_hbm.at[idx_v], sem).wait()


def _dispatch_call(flat, slots):
    mesh = plsc.VectorSubcoreMesh(core_axis_name="c", subcore_axis_name="s",
                                  num_cores=SC_CORES, num_subcores=SC_SUBCORES)
    fn = pl.kernel(
        _dispatch_body,
        mesh=mesh,
        out_type=jax.ShapeDtypeStruct((NROWS, HID), jnp.float32),
        scratch_types=[
            pltpu.VMEM((CHUNK,), jnp.int32),
            pltpu.VMEM((CHUNK, HID), jnp.float32),
            pltpu.SemaphoreType.DMA,
        ],
    )
    return fn(flat, slots)


FTS = 1024
FT = FF // FTS


def _ffn_body(x_ref, w1_ref, b1_ref, w2_ref, b2_ref, y_ref):
    f = pl.program_id(1)
    x = x_ref[...]
    h = lax.dot_general(x, w1_ref[0], (((1,), (1,)), ((), ())),
                        preferred_element_type=jnp.float32)
    h = h + b1_ref[0]
    h = 0.5 * h * (1.0 + lax.erf(h * jnp.float32(0.7071067811865476)))
    part = lax.dot_general(h, w2_ref[0], (((1,), (1,)), ((), ())),
                           preferred_element_type=jnp.float32)

    @pl.when(f == 0)
    def _():
        y_ref[...] = part + b2_ref[0]

    @pl.when(f != 0)
    def _():
        y_ref[...] = y_ref[...] + part


def _ffn_call(xdisp, W1, b1, W2, b2):
    return pl.pallas_call(
        _ffn_body,
        grid=(NE, FT),
        in_specs=[
            pl.BlockSpec((CAP, HID), lambda e, f: (e, 0)),
            pl.BlockSpec((1, FTS, HID), lambda e, f: (e, f, 0)),
            pl.BlockSpec((1, 1, FTS), lambda e, f: (e * FT + f, 0, 0)),
            pl.BlockSpec((1, HID, FTS), lambda e, f: (e, 0, f)),
            pl.BlockSpec((1, 1, HID), lambda e, f: (e, 0, 0)),
        ],
        out_specs=pl.BlockSpec((CAP, HID), lambda e, f: (e, 0)),
        out_shape=jax.ShapeDtypeStruct((NROWS, HID), jnp.float32),
    )(xdisp, W1, b1.reshape(NE * FT, 1, FTS), W2, b2.reshape(NE, 1, HID))


def _gather_body(y_hbm, slots_hbm, y0_hbm, y1_hbm, idx_v, rows_v, sem):
    wid = lax.axis_index("s") * SC_CORES + lax.axis_index("c")
    base = pl.multiple_of(wid * TOKS_PER_W, TOKS_PER_W)
    pltpu.sync_copy(slots_hbm.at[pl.ds(base, TOKS_PER_W)], idx_v)
    pltpu.async_copy(y_hbm.at[idx_v], rows_v, sem).wait()
    pltpu.sync_copy(rows_v, y0_hbm.at[pl.ds(base, TOKS_PER_W)])
    off1 = pl.multiple_of(TOKENS + base, TOKS_PER_W)
    pltpu.sync_copy(slots_hbm.at[pl.ds(off1, TOKS_PER_W)], idx_v)
    pltpu.async_copy(y_hbm.at[idx_v], rows_v, sem).wait()
    pltpu.sync_copy(rows_v, y1_hbm.at[pl.ds(base, TOKS_PER_W)])


def _gather_call(y, slots):
    mesh = plsc.VectorSubcoreMesh(core_axis_name="c", subcore_axis_name="s",
                                  num_cores=SC_CORES, num_subcores=SC_SUBCORES)
    fn = pl.kernel(
        _gather_body,
        mesh=mesh,
        out_type=(
            jax.ShapeDtypeStruct((TOKENS, HID), jnp.float32),
            jax.ShapeDtypeStruct((TOKENS, HID), jnp.float32),
        ),
        scratch_types=[
            pltpu.VMEM((TOKS_PER_W,), jnp.int32),
            pltpu.VMEM((TOKS_PER_W, HID), jnp.float32),
            pltpu.SemaphoreType.DMA,
        ],
    )
    return fn(y, slots)


TBLK = 256


def _combine_body(x_ref, w_ref, y0_ref, y1_ref, o_ref):
    w0 = w_ref[:, 0:1]
    w1 = w_ref[:, 1:2]
    o_ref[...] = x_ref[...] + w0 * y0_ref[...] + w1 * y1_ref[...]


def _combine_call(x2, w2d, y0, y1):
    return pl.pallas_call(
        _combine_body,
        grid=(TOKENS // TBLK,),
        in_specs=[
            pl.BlockSpec((TBLK, HID), lambda t: (t, 0)),
            pl.BlockSpec((TBLK, 2), lambda t: (t, 0)),
            pl.BlockSpec((TBLK, HID), lambda t: (t, 0)),
            pl.BlockSpec((TBLK, HID), lambda t: (t, 0)),
        ],
        out_specs=pl.BlockSpec((TBLK, HID), lambda t: (t, 0)),
        out_shape=jax.ShapeDtypeStruct((TOKENS, HID), jnp.float32),
    )(x2, w2d, y0, y1)


@jax.jit
def kernel(x, gamma, beta, wg, W1, b1, W2, b2):
    x2 = x.reshape(TOKENS, HID)
    flat, slots2, w2d, aux, counts = _routing_call(
        x2, gamma.reshape(1, HID), beta.reshape(1, HID), wg.T)
    slots = jnp.concatenate([slots2[:, 0], slots2[:, 1]])
    xdisp = _dispatch_call(flat, slots)
    y = _ffn_call(xdisp, W1, b1, W2, b2)
    y0, y1 = _gather_call(y, slots)
    out = _combine_call(x2, w2d, y0, y1)
    return out.reshape(x.shape), aux[0, 0], counts[0]
